# speedup vs baseline: 1.3526x; 1.3526x over previous
import functools

import jax
import jax.numpy as jnp
from jax import lax
from jax.experimental import pallas as pl
from jax.experimental.pallas import tpu as pltpu
from jax.experimental.pallas import tpu_sc as plsc

_VOCAB = 49152
_HIDDEN = 2048
_BATCH = 4
_SEQ = 4096
_NROWS = _BATCH * _SEQ
_NW = 32
_ROWS_PER_W = _NROWS // _NW
_CHUNK = 16
_NBUF = 2
_NCHUNK = _ROWS_PER_W // _CHUNK
_NGROUP = _NCHUNK // _NBUF


def _emb_lookup(idx, table):
    mesh = plsc.VectorSubcoreMesh(core_axis_name="c", subcore_axis_name="s")

    @functools.partial(
        pl.kernel,
        mesh=mesh,
        out_type=jax.ShapeDtypeStruct((_NROWS, _HIDDEN), jnp.float32),
        scratch_types=[
            pltpu.VMEM((_ROWS_PER_W,), jnp.int32),
            pltpu.VMEM((_CHUNK, _HIDDEN), jnp.float32),
            pltpu.VMEM((_CHUNK, _HIDDEN), jnp.float32),
            pltpu.SemaphoreType.DMA,
            pltpu.SemaphoreType.DMA,
        ],
    )
    def body(idx_hbm, table_hbm, out_hbm, idx_v, buf0, buf1, sem0, sem1):
        wid = lax.axis_index("s") * 2 + lax.axis_index("c")
        base = wid * _ROWS_PER_W
        pltpu.sync_copy(idx_hbm.at[pl.ds(base, _ROWS_PER_W)], idx_v)
        bufs = (buf0, buf1)
        sems = (sem0, sem1)

        def gather(chunk, b):
            return pltpu.make_async_copy(
                table_hbm.at[idx_v.at[pl.ds(chunk * _CHUNK, _CHUNK)]],
                bufs[b], sems[b])

        def drain(chunk, b):
            gather(chunk, b).wait()
            pltpu.sync_copy(
                bufs[b], out_hbm.at[pl.ds(base + chunk * _CHUNK, _CHUNK)])

        for b in range(_NBUF):
            gather(b, b).start()

        def group_body(g, carry):
            for b in range(_NBUF):
                chunk = g * _NBUF + b
                drain(chunk, b)
                gather(chunk + _NBUF, b).start()
            return carry

        lax.fori_loop(0, _NGROUP - 1, group_body, 0)

        for b in range(_NBUF):
            drain((_NGROUP - 1) * _NBUF + b, b)

    return body(idx, table)


def kernel(input_ids, input_mask, token_embedding_weight):
    del input_mask
    idx = jnp.transpose(input_ids, (1, 0)).reshape(_NROWS).astype(jnp.int32)
    out = _emb_lookup(idx, token_embedding_weight)
    return out.reshape(_SEQ, _BATCH, _HIDDEN)


# speedup vs baseline: 2.9942x; 2.2137x over previous
import functools

import jax
import jax.numpy as jnp
from jax import lax
from jax.experimental import pallas as pl
from jax.experimental.pallas import tpu as pltpu
from jax.experimental.pallas import tpu_sc as plsc

_VOCAB = 49152
_HIDDEN = 2048
_BATCH = 4
_SEQ = 4096
_NROWS = _BATCH * _SEQ
_NW = 32
_ROWS_PER_W = _NROWS // _NW
_CHUNK = 16
_NBUF = 2
_NCHUNK = _ROWS_PER_W // _CHUNK
_NGROUP = _NCHUNK // _NBUF


def _emb_lookup(idx, table):
    mesh = plsc.VectorSubcoreMesh(core_axis_name="c", subcore_axis_name="s")

    @functools.partial(
        pl.kernel,
        mesh=mesh,
        out_type=jax.ShapeDtypeStruct((_SEQ, _BATCH, _HIDDEN), jnp.float32),
        scratch_types=[
            pltpu.VMEM((_ROWS_PER_W,), jnp.int32),
            pltpu.VMEM((_CHUNK, _HIDDEN), jnp.float32),
            pltpu.VMEM((_CHUNK, _HIDDEN), jnp.float32),
            pltpu.SemaphoreType.DMA,
            pltpu.SemaphoreType.DMA,
        ],
    )
    def body(idx_hbm, table_hbm, out_hbm, idx_v, buf0, buf1, sem0, sem1):
        wid = lax.axis_index("s") * 2 + lax.axis_index("c")
        base = wid * _ROWS_PER_W
        s_base = base // _BATCH
        s_per_chunk = _CHUNK // _BATCH
        pltpu.sync_copy(idx_hbm.at[pl.ds(base, _ROWS_PER_W)], idx_v)
        bufs = (buf0, buf1)
        sems = (sem0, sem1)

        def gather(chunk, b):
            return pltpu.make_async_copy(
                table_hbm.at[idx_v.at[pl.ds(chunk * _CHUNK, _CHUNK)]],
                bufs[b], sems[b])

        def drain(chunk, b):
            gather(chunk, b).wait()
            s_off = s_base + chunk * s_per_chunk
            for j in range(s_per_chunk):
                pltpu.sync_copy(
                    bufs[b].at[pl.ds(j * _BATCH, _BATCH)],
                    out_hbm.at[s_off + j])

        for b in range(_NBUF):
            gather(b, b).start()

        def group_body(g, carry):
            for b in range(_NBUF):
                chunk = g * _NBUF + b
                drain(chunk, b)
                gather(chunk + _NBUF, b).start()
            return carry

        lax.fori_loop(0, _NGROUP - 1, group_body, 0)

        for b in range(_NBUF):
            drain((_NGROUP - 1) * _NBUF + b, b)

    return body(idx, table)


def kernel(input_ids, input_mask, token_embedding_weight):
    del input_mask
    idx = jnp.transpose(input_ids, (1, 0)).reshape(_NROWS).astype(jnp.int32)
    return _emb_lookup(idx, token_embedding_weight)


# speedup vs baseline: 2.9976x; 1.0012x over previous
import functools

import jax
import jax.numpy as jnp
from jax import lax
from jax.experimental import pallas as pl
from jax.experimental.pallas import tpu as pltpu
from jax.experimental.pallas import tpu_sc as plsc

_VOCAB = 49152
_HIDDEN = 2048
_BATCH = 4
_SEQ = 4096
_NROWS = _BATCH * _SEQ
_NW = 32
_ROWS_PER_W = _NROWS // _NW
_CHUNK = 8
_NBUF = 4
_NCHUNK = _ROWS_PER_W // _CHUNK
_NGROUP = _NCHUNK // _NBUF
_SPC = _CHUNK // _BATCH


def _emb_lookup(idx, table):
    mesh = plsc.VectorSubcoreMesh(core_axis_name="c", subcore_axis_name="s")

    @functools.partial(
        pl.kernel,
        mesh=mesh,
        out_type=jax.ShapeDtypeStruct((_SEQ, _BATCH, _HIDDEN), jnp.float32),
        scratch_types=[
            pltpu.VMEM((_ROWS_PER_W,), jnp.int32),
        ]
        + [pltpu.VMEM((_CHUNK, _HIDDEN), jnp.float32) for _ in range(_NBUF)]
        + [pltpu.SemaphoreType.DMA for _ in range(2 * _NBUF)],
    )
    def body(idx_hbm, table_hbm, out_hbm, idx_v, *scratch):
        bufs = scratch[:_NBUF]
        gsems = scratch[_NBUF:2 * _NBUF]
        ssems = scratch[2 * _NBUF:]
        wid = lax.axis_index("s") * 2 + lax.axis_index("c")
        base = wid * _ROWS_PER_W
        s_base = base // _BATCH
        pltpu.sync_copy(idx_hbm.at[pl.ds(base, _ROWS_PER_W)], idx_v)

        def g_copy(chunk, b):
            return pltpu.make_async_copy(
                table_hbm.at[idx_v.at[pl.ds(chunk * _CHUNK, _CHUNK)]],
                bufs[b], gsems[b])

        def s_copy(chunk, b, j):
            return pltpu.make_async_copy(
                bufs[b].at[pl.ds(j * _BATCH, _BATCH)],
                out_hbm.at[s_base + chunk * _SPC + j], ssems[b])

        def step(chunk, b, wait_prev_scatter, prefetch):
            g_copy(chunk, b).wait()
            for j in range(_SPC):
                s_copy(chunk, b, j).start()
            nb = (b + 2) % _NBUF
            if wait_prev_scatter:
                for j in range(_SPC):
                    s_copy(chunk - 2, nb, j).wait()
            if prefetch:
                g_copy(chunk + 2, nb).start()

        g_copy(0, 0).start()
        g_copy(1, 1).start()

        step(0, 0, False, True)
        step(1, 1, False, True)
        step(2, 2, True, True)
        step(3, 3, True, True)

        def group_body(g, carry):
            c0 = g * _NBUF
            for b in range(_NBUF):
                step(c0 + b, b, True, True)
            return carry

        lax.fori_loop(1, _NGROUP - 1, group_body, 0)

        c0 = (_NGROUP - 1) * _NBUF
        step(c0 + 0, 0, True, True)
        step(c0 + 1, 1, True, True)
        step(c0 + 2, 2, True, False)
        step(c0 + 3, 3, True, False)
        for b, chunk in ((2, _NCHUNK - 2), (3, _NCHUNK - 1)):
            for j in range(_SPC):
                s_copy(chunk, b, j).wait()

    return body(idx, table)


def kernel(input_ids, input_mask, token_embedding_weight):
    del input_mask
    idx = jnp.transpose(input_ids, (1, 0)).reshape(_NROWS).astype(jnp.int32)
    return _emb_lookup(idx, token_embedding_weight)
